# baseline (speedup 1.0000x reference)
import jax
import jax.numpy as jnp
from jax.experimental import pallas as pl
from jax.experimental.pallas import tpu as pltpu

_BLOCK = 1000


def _fused_kernel(x_ref, wzn_ref, wh_ref, bzn_ref, bh_ref, wlin_ref, blin_ref,
                  out_ref):
    x = x_ref[...]
    zl = jax.lax.dot_general(wzn_ref[...], x, (((0,), (1,)), ((), ())),
                             preferred_element_type=jnp.float32)
    hl = jax.lax.dot_general(wh_ref[...], x, (((0,), (1,)), ((), ())),
                             preferred_element_type=jnp.float32)
    s = jax.nn.sigmoid(zl + bzn_ref[...])
    t = jnp.tanh(hl + bh_ref[...])
    h = jax.nn.relu(s * t)
    o = jax.lax.dot_general(wlin_ref[...], h, (((0,), (0,)), ((), ())),
                            preferred_element_type=jnp.float32)
    out_ref[0] = o + blin_ref[...]


def kernel(x, edge_index, edge_weight, W_xz, b_xz, W_hz, b_hz, W_xr, b_xr,
           W_hr, b_hr, W_xh, b_xh, W_hh, b_hh, W_lin, b_lin):
    n, f_in = x.shape
    hid = W_xz.shape[1]
    wzn = -W_xz
    bzn = -(b_xz + b_hz).reshape(hid, 1)
    bh = (b_xh + b_hh).reshape(hid, 1)
    blin = b_lin.reshape(1, 1)

    grid = n // _BLOCK
    rep = lambda i: (0, 0)
    out_row = pl.pallas_call(
        _fused_kernel,
        grid=(grid,),
        in_specs=[
            pl.BlockSpec((_BLOCK, f_in), lambda i: (i, 0)),
            pl.BlockSpec((f_in, hid), rep),
            pl.BlockSpec((f_in, hid), rep),
            pl.BlockSpec((hid, 1), rep),
            pl.BlockSpec((hid, 1), rep),
            pl.BlockSpec((hid, 1), rep),
            pl.BlockSpec((1, 1), rep),
        ],
        out_specs=pl.BlockSpec((1, 1, _BLOCK), lambda i: (i, 0, 0)),
        out_shape=jax.ShapeDtypeStruct((grid, 1, _BLOCK), x.dtype),
        compiler_params=pltpu.CompilerParams(
            dimension_semantics=("arbitrary",),
        ),
    )(x, wzn, W_xh, bzn, bh, W_lin, blin)
    return out_row.reshape(n, 1)


# speedup vs baseline: 1.3028x; 1.3028x over previous
import jax
import jax.numpy as jnp
from jax.experimental import pallas as pl
from jax.experimental.pallas import tpu as pltpu

_BLOCK = 5000


def _fused_kernel(x_ref, wzn_ref, wh_ref, bzn_ref, bh_ref, wlin_ref, blin_ref,
                  out_ref):
    x = x_ref[...]
    zl = jax.lax.dot_general(wzn_ref[...], x, (((0,), (1,)), ((), ())),
                             preferred_element_type=jnp.float32)
    hl = jax.lax.dot_general(wh_ref[...], x, (((0,), (1,)), ((), ())),
                             preferred_element_type=jnp.float32)
    s = jax.nn.sigmoid(zl + bzn_ref[...])
    t = jnp.tanh(hl + bh_ref[...])
    h = jax.nn.relu(s * t)
    o = jax.lax.dot_general(wlin_ref[...], h, (((0,), (0,)), ((), ())),
                            preferred_element_type=jnp.float32)
    out_ref[0] = o + blin_ref[...]


def kernel(x, edge_index, edge_weight, W_xz, b_xz, W_hz, b_hz, W_xr, b_xr,
           W_hr, b_hr, W_xh, b_xh, W_hh, b_hh, W_lin, b_lin):
    n, f_in = x.shape
    hid = W_xz.shape[1]
    wzn = -W_xz
    bzn = -(b_xz + b_hz).reshape(hid, 1)
    bh = (b_xh + b_hh).reshape(hid, 1)
    blin = b_lin.reshape(1, 1)

    grid = n // _BLOCK
    rep = lambda i: (0, 0)
    out_row = pl.pallas_call(
        _fused_kernel,
        grid=(grid,),
        in_specs=[
            pl.BlockSpec((_BLOCK, f_in), lambda i: (i, 0)),
            pl.BlockSpec((f_in, hid), rep),
            pl.BlockSpec((f_in, hid), rep),
            pl.BlockSpec((hid, 1), rep),
            pl.BlockSpec((hid, 1), rep),
            pl.BlockSpec((hid, 1), rep),
            pl.BlockSpec((1, 1), rep),
        ],
        out_specs=pl.BlockSpec((1, 1, _BLOCK), lambda i: (i, 0, 0)),
        out_shape=jax.ShapeDtypeStruct((grid, 1, _BLOCK), x.dtype),
        compiler_params=pltpu.CompilerParams(
            dimension_semantics=("arbitrary",),
        ),
    )(x, wzn, W_xh, bzn, bh, W_lin, blin)
    return out_row.reshape(n, 1)
